# baseline (speedup 1.0000x reference)
import jax
import jax.numpy as jnp
from jax import lax
from jax.experimental import pallas as pl
from jax.experimental.pallas import tpu as pltpu
from jax.experimental.pallas import tpu_sc as plsc

_NUM_FIELDS = 26
_VOCAB = 100000
_EMBED = 16
_BATCH = 16384

_NC, _NS, _LANES = 2, 16, 16
_NW = _NC * _NS
_NROWS = _NUM_FIELDS * _EMBED
_RPW = _NROWS // _NW
_UNROLL = 16
_GROUPS = _BATCH // _LANES // _UNROLL


def _gather_row(row_v, io_v):
    @plsc.parallel_loop(0, _BATCH, step=_LANES * _UNROLL)
    def step(base):
        for u in range(_UNROLL):
            b = base + u * _LANES
            vidx = plsc.bitcast(io_v[pl.ds(b, _LANES)], jnp.int32)
            io_v[pl.ds(b, _LANES)] = plsc.load_gather(row_v, [vidx])


def _body(idx_hbm, tab_hbm, out_hbm, row_v, io_v, sem_r, sem_o):
    wid = lax.axis_index("s") * _NC + lax.axis_index("c")
    r0 = wid * _RPW

    pltpu.async_copy(tab_hbm.at[r0 // _EMBED, r0 % _EMBED], row_v, sem_r)
    pltpu.sync_copy(idx_hbm.at[r0 // _EMBED], io_v)

    def wait_row():
        pltpu.make_async_copy(tab_hbm.at[0, 0], row_v, sem_r).wait()

    def wait_out(r):
        pltpu.make_async_copy(io_v, out_hbm.at[r], sem_o).wait()

    def do_row(i, carry):
        r = r0 + i
        rn = r + 1
        wait_row()
        _gather_row(row_v, io_v)
        pltpu.async_copy(tab_hbm.at[rn // _EMBED, rn % _EMBED], row_v, sem_r)
        pltpu.async_copy(io_v, out_hbm.at[r], sem_o)
        wait_out(r)
        pltpu.sync_copy(idx_hbm.at[rn // _EMBED], io_v)
        return carry

    lax.fori_loop(0, _RPW - 1, do_row, 0)

    r_last = r0 + _RPW - 1
    wait_row()
    _gather_row(row_v, io_v)
    pltpu.async_copy(io_v, out_hbm.at[r_last], sem_o)
    wait_out(r_last)


_mesh = plsc.VectorSubcoreMesh(
    core_axis_name="c", subcore_axis_name="s",
    num_cores=_NC, num_subcores=_NS)

_launch = pl.kernel(
    _body,
    out_type=jax.ShapeDtypeStruct((_NROWS, _BATCH), jnp.float32),
    mesh=_mesh,
    scratch_types=[
        pltpu.VMEM((_VOCAB,), jnp.float32),
        pltpu.VMEM((_BATCH,), jnp.float32),
        pltpu.SemaphoreType.DMA,
        pltpu.SemaphoreType.DMA,
    ],
    compiler_params=pltpu.CompilerParams(needs_layout_passes=False),
)


@jax.jit
def kernel(indices, tables):
    tab_t = tables.transpose(0, 2, 1)
    idx_t = lax.bitcast_convert_type(indices.T, jnp.float32)
    out_t = _launch(idx_t, tab_t)
    return out_t.T


# speedup vs baseline: 1.0126x; 1.0126x over previous
import jax
import jax.numpy as jnp
from jax import lax
from jax.experimental import pallas as pl
from jax.experimental.pallas import tpu as pltpu
from jax.experimental.pallas import tpu_sc as plsc

_NUM_FIELDS = 26
_VOCAB = 100000
_EMBED = 16
_BATCH = 16384

_NC, _NS, _LANES = 2, 16, 16
_NW = _NC * _NS
_NROWS = _NUM_FIELDS * _EMBED
_RPW = _NROWS // _NW
_UNROLL = 8
_HV = 49920
_HV2 = 50048
_TAIL = _VOCAB - _HV - _HV2


def _gather_row(row_v, io_v):
    @plsc.parallel_loop(0, _BATCH, step=_LANES * _UNROLL)
    def step(base):
        for u in range(_UNROLL):
            b = base + u * _LANES
            vidx = plsc.bitcast(io_v[pl.ds(b, _LANES)], jnp.int32)
            io_v[pl.ds(b, _LANES)] = plsc.load_gather(row_v, [vidx])


def _body(idx_hbm, tab_hbm, tails_hbm, out_hbm, row_v, io_v,
          sem_r, sem_r2, sem_t, sem_o):
    wid = lax.axis_index("s") * _NC + lax.axis_index("c")
    r0 = wid * _RPW

    def fire_row(r):
        t_row = tab_hbm.at[r // _EMBED, r % _EMBED]
        pltpu.async_copy(t_row.at[pl.ds(0, _HV)],
                         row_v.at[pl.ds(0, _HV)], sem_r)
        pltpu.async_copy(t_row.at[pl.ds(_HV, _HV2)],
                         row_v.at[pl.ds(_HV, _HV2)], sem_r2)
        pltpu.async_copy(tails_hbm.at[pl.ds(r * _TAIL, _TAIL)],
                         row_v.at[pl.ds(_HV + _HV2, _TAIL)], sem_t)

    fire_row(r0)
    pltpu.sync_copy(idx_hbm.at[r0 // _EMBED], io_v)

    def wait_row():
        t_row = tab_hbm.at[0, 0]
        pltpu.make_async_copy(t_row.at[pl.ds(0, _HV)],
                              row_v.at[pl.ds(0, _HV)], sem_r).wait()
        pltpu.make_async_copy(t_row.at[pl.ds(_HV, _HV2)],
                              row_v.at[pl.ds(_HV, _HV2)], sem_r2).wait()
        pltpu.make_async_copy(tails_hbm.at[pl.ds(0, _TAIL)],
                              row_v.at[pl.ds(_HV + _HV2, _TAIL)],
                              sem_t).wait()

    def wait_out(r):
        pltpu.make_async_copy(io_v, out_hbm.at[r], sem_o).wait()

    def do_row(i, carry):
        r = r0 + i
        rn = r + 1
        wait_row()
        _gather_row(row_v, io_v)
        fire_row(rn)
        pltpu.async_copy(io_v, out_hbm.at[r], sem_o)
        wait_out(r)
        pltpu.sync_copy(idx_hbm.at[rn // _EMBED], io_v)
        return carry

    lax.fori_loop(0, _RPW - 1, do_row, 0)

    r_last = r0 + _RPW - 1
    wait_row()
    _gather_row(row_v, io_v)
    pltpu.async_copy(io_v, out_hbm.at[r_last], sem_o)
    wait_out(r_last)


_mesh = plsc.VectorSubcoreMesh(
    core_axis_name="c", subcore_axis_name="s",
    num_cores=_NC, num_subcores=_NS)

_launch = pl.kernel(
    _body,
    out_type=jax.ShapeDtypeStruct((_NROWS, _BATCH), jnp.float32),
    mesh=_mesh,
    scratch_types=[
        pltpu.VMEM((_VOCAB,), jnp.float32),
        pltpu.VMEM((_BATCH,), jnp.float32),
        pltpu.SemaphoreType.DMA,
        pltpu.SemaphoreType.DMA,
        pltpu.SemaphoreType.DMA,
        pltpu.SemaphoreType.DMA,
    ],
    compiler_params=pltpu.CompilerParams(needs_layout_passes=False),
)


@jax.jit
def kernel(indices, tables):
    tab_t = tables.transpose(0, 2, 1)
    idx_t = lax.bitcast_convert_type(indices.T, jnp.float32)
    tails = tab_t[:, :, _HV + _HV2:].reshape(-1)
    out_t = _launch(idx_t, tab_t, tails)
    return out_t.T


# speedup vs baseline: 1.0164x; 1.0037x over previous
import jax
import jax.numpy as jnp
from jax import lax
from jax.experimental import pallas as pl
from jax.experimental.pallas import tpu as pltpu
from jax.experimental.pallas import tpu_sc as plsc

_NUM_FIELDS = 26
_VOCAB = 100000
_EMBED = 16
_BATCH = 16384

_NC, _NS, _LANES = 2, 16, 16
_NW = _NC * _NS
_NROWS = _NUM_FIELDS * _EMBED
_RPW = _NROWS // _NW
_UNROLL = 8
_HV = 49920
_HV2 = 50048
_TAIL = _VOCAB - _HV - _HV2


def _gather_row(row_v, io_v):
    @plsc.parallel_loop(0, _BATCH, step=_LANES, unroll=_UNROLL)
    def step(b):
        vidx = plsc.bitcast(io_v[pl.ds(b, _LANES)], jnp.int32)
        io_v[pl.ds(b, _LANES)] = plsc.load_gather(row_v, [vidx])


def _body(idx_hbm, tab_hbm, tails_hbm, out_hbm, row_v, io_v,
          sem_r, sem_r2, sem_t, sem_o):
    wid = lax.axis_index("s") * _NC + lax.axis_index("c")
    r0 = wid * _RPW

    def fire_row(r):
        t_row = tab_hbm.at[r // _EMBED, r % _EMBED]
        pltpu.async_copy(t_row.at[pl.ds(0, _HV)],
                         row_v.at[pl.ds(0, _HV)], sem_r)
        pltpu.async_copy(t_row.at[pl.ds(_HV, _HV2)],
                         row_v.at[pl.ds(_HV, _HV2)], sem_r2)
        pltpu.async_copy(tails_hbm.at[pl.ds(r * _TAIL, _TAIL)],
                         row_v.at[pl.ds(_HV + _HV2, _TAIL)], sem_t)

    fire_row(r0)
    pltpu.sync_copy(idx_hbm.at[r0 // _EMBED], io_v)

    def wait_row():
        t_row = tab_hbm.at[0, 0]
        pltpu.make_async_copy(t_row.at[pl.ds(0, _HV)],
                              row_v.at[pl.ds(0, _HV)], sem_r).wait()
        pltpu.make_async_copy(t_row.at[pl.ds(_HV, _HV2)],
                              row_v.at[pl.ds(_HV, _HV2)], sem_r2).wait()
        pltpu.make_async_copy(tails_hbm.at[pl.ds(0, _TAIL)],
                              row_v.at[pl.ds(_HV + _HV2, _TAIL)],
                              sem_t).wait()

    def wait_out(r):
        pltpu.make_async_copy(io_v, out_hbm.at[r], sem_o).wait()

    def do_row(i, carry):
        r = r0 + i
        rn = r + 1
        wait_row()
        _gather_row(row_v, io_v)
        fire_row(rn)
        pltpu.async_copy(io_v, out_hbm.at[r], sem_o)
        wait_out(r)
        pltpu.sync_copy(idx_hbm.at[rn // _EMBED], io_v)
        return carry

    lax.fori_loop(0, _RPW - 1, do_row, 0)

    r_last = r0 + _RPW - 1
    wait_row()
    _gather_row(row_v, io_v)
    pltpu.async_copy(io_v, out_hbm.at[r_last], sem_o)
    wait_out(r_last)


_mesh = plsc.VectorSubcoreMesh(
    core_axis_name="c", subcore_axis_name="s",
    num_cores=_NC, num_subcores=_NS)

_launch = pl.kernel(
    _body,
    out_type=jax.ShapeDtypeStruct((_NROWS, _BATCH), jnp.float32),
    mesh=_mesh,
    scratch_types=[
        pltpu.VMEM((_VOCAB,), jnp.float32),
        pltpu.VMEM((_BATCH,), jnp.float32),
        pltpu.SemaphoreType.DMA,
        pltpu.SemaphoreType.DMA,
        pltpu.SemaphoreType.DMA,
        pltpu.SemaphoreType.DMA,
    ],
    compiler_params=pltpu.CompilerParams(needs_layout_passes=False),
)


@jax.jit
def kernel(indices, tables):
    tab_t = tables.transpose(0, 2, 1)
    idx_t = lax.bitcast_convert_type(indices.T, jnp.float32)
    tails = tab_t[:, :, _HV + _HV2:].reshape(-1)
    out_t = _launch(idx_t, tab_t, tails)
    return out_t.T


# speedup vs baseline: 1.0166x; 1.0002x over previous
import jax
import jax.numpy as jnp
from jax import lax
from jax.experimental import pallas as pl
from jax.experimental.pallas import tpu as pltpu
from jax.experimental.pallas import tpu_sc as plsc

_NUM_FIELDS = 26
_VOCAB = 100000
_EMBED = 16
_BATCH = 16384

_NC, _NS, _LANES = 2, 16, 16
_NW = _NC * _NS
_NROWS = _NUM_FIELDS * _EMBED
_RPW = _NROWS // _NW
_HV = 49920
_HV2 = 50048
_TAIL = _VOCAB - _HV - _HV2


def _gather_row(row_v, io_v):
    @plsc.parallel_loop(0, _BATCH, step=_LANES, unroll=8)
    def step(b):
        vidx = plsc.bitcast(io_v[pl.ds(b, _LANES)], jnp.int32)
        io_v[pl.ds(b, _LANES)] = plsc.load_gather(row_v, [vidx])


def _body(idx_hbm, tab_hbm, tails_hbm, out_hbm, row_v, io_v,
          sem_r, sem_r2, sem_t, sem_o):
    wid = lax.axis_index("s") * _NC + lax.axis_index("c")
    r0 = wid * _RPW

    def fire_row(r):
        t_row = tab_hbm.at[r // _EMBED, r % _EMBED]
        pltpu.async_copy(t_row.at[pl.ds(0, _HV)],
                         row_v.at[pl.ds(0, _HV)], sem_r)
        pltpu.async_copy(t_row.at[pl.ds(_HV, _HV2)],
                         row_v.at[pl.ds(_HV, _HV2)], sem_r2)
        pltpu.async_copy(tails_hbm.at[pl.ds(r * _TAIL, _TAIL)],
                         row_v.at[pl.ds(_HV + _HV2, _TAIL)], sem_t)

    fire_row(r0)
    pltpu.sync_copy(idx_hbm.at[r0 // _EMBED], io_v)

    def wait_row():
        t_row = tab_hbm.at[0, 0]
        pltpu.make_async_copy(t_row.at[pl.ds(0, _HV)],
                              row_v.at[pl.ds(0, _HV)], sem_r).wait()
        pltpu.make_async_copy(t_row.at[pl.ds(_HV, _HV2)],
                              row_v.at[pl.ds(_HV, _HV2)], sem_r2).wait()
        pltpu.make_async_copy(tails_hbm.at[pl.ds(0, _TAIL)],
                              row_v.at[pl.ds(_HV + _HV2, _TAIL)],
                              sem_t).wait()

    def wait_out(r):
        pltpu.make_async_copy(io_v, out_hbm.at[r], sem_o).wait()

    def do_row(i, carry):
        r = r0 + i
        rn = r + 1
        wait_row()
        _gather_row(row_v, io_v)
        fire_row(rn)
        pltpu.async_copy(io_v, out_hbm.at[r], sem_o)
        wait_out(r)
        pltpu.sync_copy(idx_hbm.at[rn // _EMBED], io_v)
        return carry

    lax.fori_loop(0, _RPW - 1, do_row, 0)

    r_last = r0 + _RPW - 1
    wait_row()
    _gather_row(row_v, io_v)
    pltpu.async_copy(io_v, out_hbm.at[r_last], sem_o)
    wait_out(r_last)


_mesh = plsc.VectorSubcoreMesh(
    core_axis_name="c", subcore_axis_name="s",
    num_cores=_NC, num_subcores=_NS)

_launch = pl.kernel(
    _body,
    out_type=jax.ShapeDtypeStruct((_NROWS, _BATCH), jnp.float32),
    mesh=_mesh,
    scratch_types=[
        pltpu.VMEM((_VOCAB,), jnp.float32),
        pltpu.VMEM((_BATCH,), jnp.float32),
        pltpu.SemaphoreType.DMA,
        pltpu.SemaphoreType.DMA,
        pltpu.SemaphoreType.DMA,
        pltpu.SemaphoreType.DMA,
    ],
    compiler_params=pltpu.CompilerParams(needs_layout_passes=False),
)


@jax.jit
def kernel(indices, tables):
    tab_t = tables.transpose(0, 2, 1)
    idx_t = lax.bitcast_convert_type(indices.T, jnp.float32)
    tails = tab_t[:, :, _HV + _HV2:].reshape(-1)
    out_t = _launch(idx_t, tab_t, tails)
    return out_t.T
